# baseline (speedup 1.0000x reference)
import functools

import jax
import jax.numpy as jnp
import numpy as np
from jax import lax
from jax.experimental import pallas as pl
from jax.experimental.pallas import tpu as pltpu
from jax.experimental.pallas import tpu_sc as plsc

_CHUNK = 128
_NW = 32


def _gather_body(n_chunks, table_hbm, idx_hbm, oidx_hbm, out_hbm,
                 idx_v, oidx_v, rows_v, sem0, sem1):
    nc = plsc.get_sparse_core_info().num_cores
    wid = lax.axis_index("s") * nc + lax.axis_index("c")
    pltpu.sync_copy(idx_hbm.at[wid], idx_v)
    pltpu.sync_copy(oidx_hbm.at[wid], oidx_v)

    def gather(j, buf, sem):
        pltpu.make_async_copy(
            table_hbm.at[idx_v.at[j]], rows_v.at[buf], sem).start()

    def wait(j, buf, sem):
        pltpu.make_async_copy(
            table_hbm.at[idx_v.at[j]], rows_v.at[buf], sem).wait()

    def scatter(j, buf):
        pltpu.sync_copy(rows_v.at[buf], out_hbm.at[oidx_v.at[j]])

    gather(0, 0, sem0)
    gather(1, 1, sem1)

    def body(g, carry):
        c0 = 2 * g
        wait(c0, 0, sem0)
        scatter(c0, 0)

        @pl.when(c0 + 2 < n_chunks)
        def _():
            gather(c0 + 2, 0, sem0)

        wait(c0 + 1, 1, sem1)
        scatter(c0 + 1, 1)

        @pl.when(c0 + 3 < n_chunks)
        def _():
            gather(c0 + 3, 1, sem1)

        return carry

    lax.fori_loop(0, n_chunks // 2, body, 0)


def _sc_gather(table, idx, oidx, n_rows, E):
    nw, n_chunks, _ = idx.shape
    mesh = plsc.VectorSubcoreMesh(core_axis_name="c", subcore_axis_name="s")
    return pl.kernel(
        functools.partial(_gather_body, n_chunks),
        out_type=jax.ShapeDtypeStruct((n_rows, E), jnp.float32),
        mesh=mesh,
        scratch_types=[
            pltpu.VMEM((n_chunks, _CHUNK), jnp.int32),
            pltpu.VMEM((n_chunks, _CHUNK), jnp.int32),
            pltpu.VMEM((2, _CHUNK, E), jnp.float32),
            pltpu.SemaphoreType.DMA,
            pltpu.SemaphoreType.DMA,
        ],
        compiler_params=pltpu.CompilerParams(use_tc_tiling_on_sc=False),
    )(table, idx, oidx)


@functools.lru_cache(maxsize=None)
def _time_major_oidx(B, T):
    j = np.arange(B * T, dtype=np.int64)
    b, t = j // T, j % T
    oidx = (t * B + b).astype(np.int32)
    return oidx.reshape(_NW, (B * T) // (_NW * _CHUNK), _CHUNK)


def _rnn_step(e_ref, wih_ref, whh_ref, bh_ref, wun_ref, bun_ref,
              y_ref, s_ref):
    t = pl.program_id(0)

    @pl.when(t == 0)
    def _():
        s_ref[...] = jnp.zeros_like(s_ref)

    e_t = e_ref[0]
    s = s_ref[...]
    h = (jnp.dot(e_t, wih_ref[...], preferred_element_type=jnp.float32)
         + jnp.dot(s, whh_ref[...], preferred_element_type=jnp.float32)
         + bh_ref[...])
    s = jnp.tanh(h)
    s_ref[...] = s
    y_ref[0] = (jnp.dot(s, wun_ref[...], preferred_element_type=jnp.float32)
                + bun_ref[...])


def _rnn_unembed(e, wih_t, whh_t, bh, wun_t, bun, *, interpret=False):
    T, B, E = e.shape
    A = wun_t.shape[1]
    return pl.pallas_call(
        _rnn_step,
        grid=(T,),
        in_specs=[
            pl.BlockSpec((1, B, E), lambda t: (t, 0, 0)),
            pl.BlockSpec((E, E), lambda t: (0, 0)),
            pl.BlockSpec((E, E), lambda t: (0, 0)),
            pl.BlockSpec((1, E), lambda t: (0, 0)),
            pl.BlockSpec((E, A), lambda t: (0, 0)),
            pl.BlockSpec((1, A), lambda t: (0, 0)),
        ],
        out_specs=pl.BlockSpec((1, B, A), lambda t: (t, 0, 0)),
        out_shape=jax.ShapeDtypeStruct((T, B, A), jnp.float32),
        scratch_shapes=[pltpu.VMEM((B, E), jnp.float32)],
        compiler_params=pltpu.CompilerParams(
            dimension_semantics=("arbitrary",),
        ),
        interpret=interpret,
    )(e, wih_t, whh_t, bh, wun_t, bun)


def kernel(x, trainable, embed_table, W_ih, W_hh, b_h, W_un, b_un):
    B, T = x.shape
    E = embed_table.shape[1]
    A = W_un.shape[0]
    n_rows = B * T
    idx = x.reshape(_NW, n_rows // (_NW * _CHUNK), _CHUNK)
    oidx = jnp.asarray(_time_major_oidx(B, T))
    e = _sc_gather(embed_table, idx, oidx, n_rows, E).reshape(T, B, E)
    y = _rnn_unembed(e, W_ih.T, W_hh.T, b_h.reshape(1, E),
                     W_un.T, b_un.reshape(1, A))
    return y.reshape(B, T, A)


# speedup vs baseline: 1.2895x; 1.2895x over previous
import functools

import jax
import jax.numpy as jnp
import numpy as np
from jax import lax
from jax.experimental import pallas as pl
from jax.experimental.pallas import tpu as pltpu
from jax.experimental.pallas import tpu_sc as plsc

_CHUNK = 128
_NW = 32


def _gather_body(n_chunks, table_hbm, idx_hbm, oidx_hbm, out_hbm,
                 idx_v, oidx_v, rows_v, sem0, sem1):
    nc = plsc.get_sparse_core_info().num_cores
    wid = lax.axis_index("s") * nc + lax.axis_index("c")
    pltpu.sync_copy(idx_hbm.at[wid], idx_v)
    pltpu.sync_copy(oidx_hbm.at[wid], oidx_v)

    def gather(j, buf, sem):
        pltpu.make_async_copy(
            table_hbm.at[idx_v.at[j]], rows_v.at[buf], sem).start()

    def wait(j, buf, sem):
        pltpu.make_async_copy(
            table_hbm.at[idx_v.at[j]], rows_v.at[buf], sem).wait()

    def scatter(j, buf):
        pltpu.sync_copy(rows_v.at[buf], out_hbm.at[oidx_v.at[j]])

    gather(0, 0, sem0)
    gather(1, 1, sem1)

    def body(g, carry):
        c0 = 2 * g
        wait(c0, 0, sem0)
        scatter(c0, 0)

        @pl.when(c0 + 2 < n_chunks)
        def _():
            gather(c0 + 2, 0, sem0)

        wait(c0 + 1, 1, sem1)
        scatter(c0 + 1, 1)

        @pl.when(c0 + 3 < n_chunks)
        def _():
            gather(c0 + 3, 1, sem1)

        return carry

    lax.fori_loop(0, n_chunks // 2, body, 0)


def _sc_gather(table, idx, oidx, n_rows, E):
    nw, n_chunks, _ = idx.shape
    mesh = plsc.VectorSubcoreMesh(core_axis_name="c", subcore_axis_name="s")
    return pl.kernel(
        functools.partial(_gather_body, n_chunks),
        out_type=jax.ShapeDtypeStruct((n_rows, E), jnp.float32),
        mesh=mesh,
        scratch_types=[
            pltpu.VMEM((n_chunks, _CHUNK), jnp.int32),
            pltpu.VMEM((n_chunks, _CHUNK), jnp.int32),
            pltpu.VMEM((2, _CHUNK, E), jnp.float32),
            pltpu.SemaphoreType.DMA,
            pltpu.SemaphoreType.DMA,
        ],
        compiler_params=pltpu.CompilerParams(use_tc_tiling_on_sc=False),
    )(table, idx, oidx)


@functools.lru_cache(maxsize=None)
def _time_major_oidx(B, T):
    j = np.arange(B * T, dtype=np.int64)
    b, t = j // T, j % T
    oidx = (2 * ((t // 2) * B + b) + t % 2).astype(np.int32)
    return oidx.reshape(_NW, (B * T) // (_NW * _CHUNK), _CHUNK)


def _permute_body(n_chunks, y_hbm, oidx_hbm, out_hbm, oidx_v, rows_v,
                  sem0, sem1):
    nc = plsc.get_sparse_core_info().num_cores
    wid = lax.axis_index("s") * nc + lax.axis_index("c")
    rows_per_w = n_chunks * _CHUNK
    base = wid * rows_per_w
    pltpu.sync_copy(oidx_hbm.at[wid], oidx_v)

    def load(j, buf, sem):
        pltpu.make_async_copy(
            y_hbm.at[pl.ds(base + j * _CHUNK, _CHUNK)], rows_v.at[buf],
            sem).start()

    def wait(j, buf, sem):
        pltpu.make_async_copy(
            y_hbm.at[pl.ds(base + j * _CHUNK, _CHUNK)], rows_v.at[buf],
            sem).wait()

    def scatter(j, buf):
        pltpu.sync_copy(rows_v.at[buf], out_hbm.at[oidx_v.at[j]])

    load(0, 0, sem0)
    load(1, 1, sem1)

    def body(g, carry):
        c0 = 2 * g
        wait(c0, 0, sem0)
        scatter(c0, 0)

        @pl.when(c0 + 2 < n_chunks)
        def _():
            load(c0 + 2, 0, sem0)

        wait(c0 + 1, 1, sem1)
        scatter(c0 + 1, 1)

        @pl.when(c0 + 3 < n_chunks)
        def _():
            load(c0 + 3, 1, sem1)

        return carry

    lax.fori_loop(0, n_chunks // 2, body, 0)


def _sc_permute(y, oidx, n_rows, A):
    nw, n_chunks, _ = oidx.shape
    mesh = plsc.VectorSubcoreMesh(core_axis_name="c", subcore_axis_name="s")
    return pl.kernel(
        functools.partial(_permute_body, n_chunks),
        out_type=jax.ShapeDtypeStruct((n_rows, A), jnp.float32),
        mesh=mesh,
        scratch_types=[
            pltpu.VMEM((n_chunks, _CHUNK), jnp.int32),
            pltpu.VMEM((2, _CHUNK, A), jnp.float32),
            pltpu.SemaphoreType.DMA,
            pltpu.SemaphoreType.DMA,
        ],
        compiler_params=pltpu.CompilerParams(use_tc_tiling_on_sc=False),
    )(y, oidx)


@functools.lru_cache(maxsize=None)
def _unscramble_oidx(B, T):
    r = np.arange(B * T, dtype=np.int64)
    b, t = r // T, r % T
    oidx = (t * B + b).astype(np.int32)
    return oidx.reshape(_NW, (B * T) // (_NW * _CHUNK), _CHUNK)


def _rnn_step(e_ref, w1e_ref, w1o_ref, whh_ref, bh_ref, wun_ref, bun_ref,
              y_ref, s_ref):
    tp = pl.program_id(0)
    nb = pl.program_id(1)
    cb = y_ref.shape[3]
    base = nb * cb

    @pl.when(tp == 0)
    def _():
        s_ref[pl.ds(base, cb)] = jnp.zeros((cb, s_ref.shape[1]), jnp.float32)

    ep = e_ref[0]
    s = s_ref[pl.ds(base, cb)]
    bh = bh_ref[...]
    bun = bun_ref[...]
    whh = whh_ref[...]
    wun = wun_ref[...]
    s = jnp.tanh(jnp.dot(ep, w1e_ref[...], preferred_element_type=jnp.float32)
                 + jnp.dot(s, whh, preferred_element_type=jnp.float32) + bh)
    y_ref[0, 0, 0] = (jnp.dot(s, wun, preferred_element_type=jnp.float32)
                      + bun)
    s = jnp.tanh(jnp.dot(ep, w1o_ref[...], preferred_element_type=jnp.float32)
                 + jnp.dot(s, whh, preferred_element_type=jnp.float32) + bh)
    s_ref[pl.ds(base, cb)] = s
    y_ref[0, 1, 0] = (jnp.dot(s, wun, preferred_element_type=jnp.float32)
                      + bun)


def _rnn_unembed(e2, w1e, w1o, whh_t, bh, wun_t, bun, *, interpret=False):
    TP, B, E2 = e2.shape
    E = E2 // 2
    A = wun_t.shape[1]
    CB = 512
    NB = B // CB
    return pl.pallas_call(
        _rnn_step,
        grid=(TP, NB),
        in_specs=[
            pl.BlockSpec((1, CB, E2), lambda tp, nb: (tp, nb, 0)),
            pl.BlockSpec((E2, E), lambda tp, nb: (0, 0)),
            pl.BlockSpec((E2, E), lambda tp, nb: (0, 0)),
            pl.BlockSpec((E, E), lambda tp, nb: (0, 0)),
            pl.BlockSpec((1, E), lambda tp, nb: (0, 0)),
            pl.BlockSpec((E, A), lambda tp, nb: (0, 0)),
            pl.BlockSpec((1, A), lambda tp, nb: (0, 0)),
        ],
        out_specs=pl.BlockSpec((1, 2, 1, CB, A),
                               lambda tp, nb: (tp, 0, nb, 0, 0)),
        out_shape=jax.ShapeDtypeStruct((TP, 2, NB, CB, A), jnp.float32),
        scratch_shapes=[pltpu.VMEM((B, E), jnp.float32)],
        compiler_params=pltpu.CompilerParams(
            dimension_semantics=("arbitrary", "arbitrary"),
        ),
        interpret=interpret,
    )(e2, w1e, w1o, whh_t, bh, wun_t, bun)


def kernel(x, trainable, embed_table, W_ih, W_hh, b_h, W_un, b_un):
    B, T = x.shape
    E = embed_table.shape[1]
    A = W_un.shape[0]
    n_rows = B * T
    idx = x.reshape(_NW, n_rows // (_NW * _CHUNK), _CHUNK)
    oidx = jnp.asarray(_time_major_oidx(B, T))
    e = _sc_gather(embed_table, idx, oidx, n_rows, E)
    e2 = e.reshape(T // 2, B, 2 * E)
    zeros = jnp.zeros((E, E), jnp.float32)
    w1e = jnp.concatenate([W_ih.T, zeros], axis=0)
    w1o = jnp.concatenate([zeros, W_ih.T], axis=0)
    y5 = _rnn_unembed(e2, w1e, w1o, W_hh.T, b_h.reshape(1, E),
                      W_un.T, b_un.reshape(1, A))
    y_flat = y5.reshape(n_rows, A)
    out_q = _sc_permute(y_flat, jnp.asarray(_unscramble_oidx(B, T)),
                        n_rows, A)
    return jnp.transpose(out_q.reshape(T, B, A), (1, 0, 2))


# speedup vs baseline: 1.8070x; 1.4013x over previous
import functools

import jax
import jax.numpy as jnp
import numpy as np
from jax import lax
from jax.experimental import pallas as pl
from jax.experimental.pallas import tpu as pltpu
from jax.experimental.pallas import tpu_sc as plsc

_CHUNK = 128
_NW = 32


def _gather_body(n_chunks, table_hbm, idx_hbm, oidx_hbm, out_hbm,
                 idx_v, oidx_v, rows_v, sem0, sem1):
    nc = plsc.get_sparse_core_info().num_cores
    wid = lax.axis_index("s") * nc + lax.axis_index("c")
    pltpu.sync_copy(idx_hbm.at[wid], idx_v)
    pltpu.sync_copy(oidx_hbm.at[wid], oidx_v)

    def gather(j, buf, sem):
        pltpu.make_async_copy(
            table_hbm.at[idx_v.at[j]], rows_v.at[buf], sem).start()

    def wait(j, buf, sem):
        pltpu.make_async_copy(
            table_hbm.at[idx_v.at[j]], rows_v.at[buf], sem).wait()

    def scatter(j, buf):
        pltpu.sync_copy(rows_v.at[buf], out_hbm.at[oidx_v.at[j]])

    gather(0, 0, sem0)
    gather(1, 1, sem1)

    def body(g, carry):
        c0 = 2 * g
        wait(c0, 0, sem0)
        scatter(c0, 0)

        @pl.when(c0 + 2 < n_chunks)
        def _():
            gather(c0 + 2, 0, sem0)

        wait(c0 + 1, 1, sem1)
        scatter(c0 + 1, 1)

        @pl.when(c0 + 3 < n_chunks)
        def _():
            gather(c0 + 3, 1, sem1)

        return carry

    lax.fori_loop(0, n_chunks // 2, body, 0)


def _sc_gather(table, idx, oidx, n_rows, E):
    nw, n_chunks, _ = idx.shape
    mesh = plsc.VectorSubcoreMesh(core_axis_name="c", subcore_axis_name="s")
    return pl.kernel(
        functools.partial(_gather_body, n_chunks),
        out_type=jax.ShapeDtypeStruct((n_rows, E), jnp.float32),
        mesh=mesh,
        scratch_types=[
            pltpu.VMEM((n_chunks, _CHUNK), jnp.int32),
            pltpu.VMEM((n_chunks, _CHUNK), jnp.int32),
            pltpu.VMEM((2, _CHUNK, E), jnp.float32),
            pltpu.SemaphoreType.DMA,
            pltpu.SemaphoreType.DMA,
        ],
        compiler_params=pltpu.CompilerParams(use_tc_tiling_on_sc=False),
    )(table, idx, oidx)


@functools.lru_cache(maxsize=None)
def _time_major_oidx(B, T):
    j = np.arange(B * T, dtype=np.int64)
    b, t = j // T, j % T
    oidx = (2 * ((t // 2) * B + b) + t % 2).astype(np.int32)
    return oidx.reshape(_NW, (B * T) // (_NW * _CHUNK), _CHUNK)


def _permute_body(n_chunks, y_hbm, oidx_hbm, out_hbm, oidx_v, rows_v,
                  sem0, sem1):
    nc = plsc.get_sparse_core_info().num_cores
    wid = lax.axis_index("s") * nc + lax.axis_index("c")
    rows_per_w = n_chunks * _CHUNK
    base = wid * rows_per_w
    pltpu.sync_copy(oidx_hbm.at[wid], oidx_v)

    def load(j, buf, sem):
        pltpu.make_async_copy(
            y_hbm.at[pl.ds(base + j * _CHUNK, _CHUNK)], rows_v.at[buf],
            sem).start()

    def wait(j, buf, sem):
        pltpu.make_async_copy(
            y_hbm.at[pl.ds(base + j * _CHUNK, _CHUNK)], rows_v.at[buf],
            sem).wait()

    def scatter(j, buf):
        pltpu.sync_copy(rows_v.at[buf], out_hbm.at[oidx_v.at[j]])

    load(0, 0, sem0)
    load(1, 1, sem1)

    def body(g, carry):
        c0 = 2 * g
        wait(c0, 0, sem0)
        scatter(c0, 0)

        @pl.when(c0 + 2 < n_chunks)
        def _():
            load(c0 + 2, 0, sem0)

        wait(c0 + 1, 1, sem1)
        scatter(c0 + 1, 1)

        @pl.when(c0 + 3 < n_chunks)
        def _():
            load(c0 + 3, 1, sem1)

        return carry

    lax.fori_loop(0, n_chunks // 2, body, 0)


def _sc_permute(y, oidx, n_rows, A):
    nw, n_chunks, _ = oidx.shape
    mesh = plsc.VectorSubcoreMesh(core_axis_name="c", subcore_axis_name="s")
    return pl.kernel(
        functools.partial(_permute_body, n_chunks),
        out_type=jax.ShapeDtypeStruct((n_rows, A), jnp.float32),
        mesh=mesh,
        scratch_types=[
            pltpu.VMEM((n_chunks, _CHUNK), jnp.int32),
            pltpu.VMEM((2, _CHUNK, A), jnp.float32),
            pltpu.SemaphoreType.DMA,
            pltpu.SemaphoreType.DMA,
        ],
        compiler_params=pltpu.CompilerParams(use_tc_tiling_on_sc=False),
    )(y, oidx)


@functools.lru_cache(maxsize=None)
def _unscramble_oidx(B, T):
    r = np.arange(B * T, dtype=np.int64)
    b, t = r // T, r % T
    oidx = (t * B + b).astype(np.int32)
    return oidx.reshape(_NW, (B * T) // (_NW * _CHUNK), _CHUNK)


def _rnn_step(e_ref, w1e_ref, w1o_ref, whh_ref, bh_ref, wun_ref, bun_ref,
              y_ref, s_ref):
    tp = pl.program_id(0)
    nb = pl.program_id(1)
    cb = y_ref.shape[3]
    base = nb * cb

    @pl.when(tp == 0)
    def _():
        s_ref[pl.ds(base, cb)] = jnp.zeros((cb, s_ref.shape[1]), jnp.float32)

    ep = e_ref[0]
    s = s_ref[pl.ds(base, cb)]
    bh = bh_ref[...]
    bun = bun_ref[...]
    whh = whh_ref[...]
    wun = wun_ref[...]
    s = jnp.tanh(jnp.dot(ep, w1e_ref[...], preferred_element_type=jnp.float32)
                 + jnp.dot(s, whh, preferred_element_type=jnp.float32) + bh)
    y_ref[0, 0, 0] = (jnp.dot(s, wun, preferred_element_type=jnp.float32)
                      + bun)
    s = jnp.tanh(jnp.dot(ep, w1o_ref[...], preferred_element_type=jnp.float32)
                 + jnp.dot(s, whh, preferred_element_type=jnp.float32) + bh)
    s_ref[pl.ds(base, cb)] = s
    y_ref[0, 1, 0] = (jnp.dot(s, wun, preferred_element_type=jnp.float32)
                      + bun)


def _rnn_unembed(e2, w1e, w1o, whh_t, bh, wun_t, bun, *, interpret=False):
    TP, B, E2 = e2.shape
    E = E2 // 2
    A = wun_t.shape[1]
    CB = B
    NB = B // CB
    return pl.pallas_call(
        _rnn_step,
        grid=(TP, NB),
        in_specs=[
            pl.BlockSpec((1, CB, E2), lambda tp, nb: (tp, nb, 0)),
            pl.BlockSpec((E2, E), lambda tp, nb: (0, 0)),
            pl.BlockSpec((E2, E), lambda tp, nb: (0, 0)),
            pl.BlockSpec((E, E), lambda tp, nb: (0, 0)),
            pl.BlockSpec((1, E), lambda tp, nb: (0, 0)),
            pl.BlockSpec((E, A), lambda tp, nb: (0, 0)),
            pl.BlockSpec((1, A), lambda tp, nb: (0, 0)),
        ],
        out_specs=pl.BlockSpec((1, 2, 1, CB, A),
                               lambda tp, nb: (tp, 0, nb, 0, 0)),
        out_shape=jax.ShapeDtypeStruct((TP, 2, NB, CB, A), jnp.float32),
        scratch_shapes=[pltpu.VMEM((B, E), jnp.float32)],
        compiler_params=pltpu.CompilerParams(
            dimension_semantics=("arbitrary", "arbitrary"),
        ),
        interpret=interpret,
    )(e2, w1e, w1o, whh_t, bh, wun_t, bun)


def kernel(x, trainable, embed_table, W_ih, W_hh, b_h, W_un, b_un):
    B, T = x.shape
    E = embed_table.shape[1]
    A = W_un.shape[0]
    n_rows = B * T
    idx = x.reshape(_NW, n_rows // (_NW * _CHUNK), _CHUNK)
    oidx = jnp.asarray(_time_major_oidx(B, T))
    e = _sc_gather(embed_table, idx, oidx, n_rows, E)
    e2 = e.reshape(T // 2, B, 2 * E)
    zeros = jnp.zeros((E, E), jnp.float32)
    w1e = jnp.concatenate([W_ih.T, zeros], axis=0)
    w1o = jnp.concatenate([zeros, W_ih.T], axis=0)
    y5 = _rnn_unembed(e2, w1e, w1o, W_hh.T, b_h.reshape(1, E),
                      W_un.T, b_un.reshape(1, A))
    y_flat = y5.reshape(n_rows, A)
    out_q = _sc_permute(y_flat, jnp.asarray(_unscramble_oidx(B, T)),
                        n_rows, A)
    return jnp.transpose(out_q.reshape(T, B, A), (1, 0, 2))


# speedup vs baseline: 1.8096x; 1.0015x over previous
import functools

import jax
import jax.numpy as jnp
import numpy as np
from jax import lax
from jax.experimental import pallas as pl
from jax.experimental.pallas import tpu as pltpu
from jax.experimental.pallas import tpu_sc as plsc

_CHUNK = 128
_NW = 32


def _gather_body(n_chunks, table_hbm, idx_hbm, oidx_hbm, out_hbm,
                 idx_v, oidx_v, rows_v, sem0, sem1):
    nc = plsc.get_sparse_core_info().num_cores
    wid = lax.axis_index("s") * nc + lax.axis_index("c")
    pltpu.sync_copy(idx_hbm.at[wid], idx_v)
    pltpu.sync_copy(oidx_hbm.at[wid], oidx_v)

    def gather(j, buf, sem):
        pltpu.make_async_copy(
            table_hbm.at[idx_v.at[j]], rows_v.at[buf], sem).start()

    def wait(j, buf, sem):
        pltpu.make_async_copy(
            table_hbm.at[idx_v.at[j]], rows_v.at[buf], sem).wait()

    def scatter(j, buf):
        pltpu.sync_copy(rows_v.at[buf], out_hbm.at[oidx_v.at[j]])

    gather(0, 0, sem0)
    gather(1, 1, sem1)

    def body(g, carry):
        c0 = 2 * g
        wait(c0, 0, sem0)
        scatter(c0, 0)

        @pl.when(c0 + 2 < n_chunks)
        def _():
            gather(c0 + 2, 0, sem0)

        wait(c0 + 1, 1, sem1)
        scatter(c0 + 1, 1)

        @pl.when(c0 + 3 < n_chunks)
        def _():
            gather(c0 + 3, 1, sem1)

        return carry

    lax.fori_loop(0, n_chunks // 2, body, 0)


def _sc_gather(table, idx, oidx, n_rows, E):
    nw, n_chunks, _ = idx.shape
    mesh = plsc.VectorSubcoreMesh(core_axis_name="c", subcore_axis_name="s")
    return pl.kernel(
        functools.partial(_gather_body, n_chunks),
        out_type=jax.ShapeDtypeStruct((n_rows, E), jnp.float32),
        mesh=mesh,
        scratch_types=[
            pltpu.VMEM((n_chunks, _CHUNK), jnp.int32),
            pltpu.VMEM((n_chunks, _CHUNK), jnp.int32),
            pltpu.VMEM((2, _CHUNK, E), jnp.float32),
            pltpu.SemaphoreType.DMA,
            pltpu.SemaphoreType.DMA,
        ],
        compiler_params=pltpu.CompilerParams(use_tc_tiling_on_sc=False),
    )(table, idx, oidx)


@functools.lru_cache(maxsize=None)
def _time_major_oidx(B, t0, t1):
    dt = t1 - t0
    j = np.arange(B * dt, dtype=np.int64)
    b, tl = j // dt, j % dt
    oidx = (2 * ((tl // 2) * B + b) + tl % 2).astype(np.int32)
    return oidx.reshape(_NW, (B * dt) // (_NW * _CHUNK), _CHUNK)


def _permute_body(n_chunks_a, n_chunks_b, ya_hbm, yb_hbm, oa_hbm, ob_hbm,
                  out_hbm, oa_v, ob_v, rows_v, sem0, sem1):
    nc = plsc.get_sparse_core_info().num_cores
    wid = lax.axis_index("s") * nc + lax.axis_index("c")
    pltpu.sync_copy(oa_hbm.at[wid], oa_v)
    pltpu.sync_copy(ob_hbm.at[wid], ob_v)

    def run(y_hbm, oidx_v, n_chunks):
        base = wid * n_chunks * _CHUNK

        def load(j, buf, sem):
            pltpu.make_async_copy(
                y_hbm.at[pl.ds(base + j * _CHUNK, _CHUNK)], rows_v.at[buf],
                sem).start()

        def wait(j, buf, sem):
            pltpu.make_async_copy(
                y_hbm.at[pl.ds(base + j * _CHUNK, _CHUNK)], rows_v.at[buf],
                sem).wait()

        def scatter(j, buf):
            pltpu.sync_copy(rows_v.at[buf], out_hbm.at[oidx_v.at[j]])

        load(0, 0, sem0)
        load(1, 1, sem1)

        def body(g, carry):
            c0 = 2 * g
            wait(c0, 0, sem0)
            scatter(c0, 0)

            @pl.when(c0 + 2 < n_chunks)
            def _():
                load(c0 + 2, 0, sem0)

            wait(c0 + 1, 1, sem1)
            scatter(c0 + 1, 1)

            @pl.when(c0 + 3 < n_chunks)
            def _():
                load(c0 + 3, 1, sem1)

            return carry

        lax.fori_loop(0, n_chunks // 2, body, 0)

    run(ya_hbm, oa_v, n_chunks_a)
    run(yb_hbm, ob_v, n_chunks_b)


def _sc_permute(ya, yb, oidx_a, oidx_b, n_rows, A):
    nw, ca, _ = oidx_a.shape
    cb = oidx_b.shape[1]
    mesh = plsc.VectorSubcoreMesh(core_axis_name="c", subcore_axis_name="s")
    return pl.kernel(
        functools.partial(_permute_body, ca, cb),
        out_type=jax.ShapeDtypeStruct((n_rows, A), jnp.float32),
        mesh=mesh,
        scratch_types=[
            pltpu.VMEM((ca, _CHUNK), jnp.int32),
            pltpu.VMEM((cb, _CHUNK), jnp.int32),
            pltpu.VMEM((2, _CHUNK, A), jnp.float32),
            pltpu.SemaphoreType.DMA,
            pltpu.SemaphoreType.DMA,
        ],
        compiler_params=pltpu.CompilerParams(use_tc_tiling_on_sc=False),
    )(ya, yb, oidx_a, oidx_b)


@functools.lru_cache(maxsize=None)
def _unscramble_oidx(B, T, r0, r1):
    r = np.arange(r0, r1, dtype=np.int64)
    b, t = r // T, r % T
    oidx = (t * B + b).astype(np.int32)
    return oidx.reshape(_NW, (r1 - r0) // (_NW * _CHUNK), _CHUNK)


def _rnn_step(e_ref, sin_ref, w1e_ref, w1o_ref, whh_ref, bh_ref, wun_ref,
              bun_ref, y_ref, sout_ref):
    tp = pl.program_id(0)

    @pl.when(tp == 0)
    def _():
        sout_ref[...] = sin_ref[...]

    ep = e_ref[0]
    s = sout_ref[...]
    bh = bh_ref[...]
    bun = bun_ref[...]
    whh = whh_ref[...]
    wun = wun_ref[...]
    s = jnp.tanh(jnp.dot(ep, w1e_ref[...], preferred_element_type=jnp.float32)
                 + jnp.dot(s, whh, preferred_element_type=jnp.float32) + bh)
    y_ref[0, 0, 0] = (jnp.dot(s, wun, preferred_element_type=jnp.float32)
                      + bun)
    s = jnp.tanh(jnp.dot(ep, w1o_ref[...], preferred_element_type=jnp.float32)
                 + jnp.dot(s, whh, preferred_element_type=jnp.float32) + bh)
    sout_ref[...] = s
    y_ref[0, 1, 0] = (jnp.dot(s, wun, preferred_element_type=jnp.float32)
                      + bun)


def _rnn_unembed(e2, s_in, w1e, w1o, whh_t, bh, wun_t, bun, *,
                 interpret=False):
    TP, B, E2 = e2.shape
    E = E2 // 2
    A = wun_t.shape[1]
    return pl.pallas_call(
        _rnn_step,
        grid=(TP,),
        in_specs=[
            pl.BlockSpec((1, B, E2), lambda tp: (tp, 0, 0)),
            pl.BlockSpec((B, E), lambda tp: (0, 0)),
            pl.BlockSpec((E2, E), lambda tp: (0, 0)),
            pl.BlockSpec((E2, E), lambda tp: (0, 0)),
            pl.BlockSpec((E, E), lambda tp: (0, 0)),
            pl.BlockSpec((1, E), lambda tp: (0, 0)),
            pl.BlockSpec((E, A), lambda tp: (0, 0)),
            pl.BlockSpec((1, A), lambda tp: (0, 0)),
        ],
        out_specs=[
            pl.BlockSpec((1, 2, 1, B, A), lambda tp: (tp, 0, 0, 0, 0)),
            pl.BlockSpec((B, E), lambda tp: (0, 0)),
        ],
        out_shape=[
            jax.ShapeDtypeStruct((TP, 2, 1, B, A), jnp.float32),
            jax.ShapeDtypeStruct((B, E), jnp.float32),
        ],
        compiler_params=pltpu.CompilerParams(
            dimension_semantics=("arbitrary",),
        ),
        interpret=interpret,
    )(e2, s_in, w1e, w1o, whh_t, bh, wun_t, bun)


def kernel(x, trainable, embed_table, W_ih, W_hh, b_h, W_un, b_un):
    B, T = x.shape
    E = embed_table.shape[1]
    A = W_un.shape[0]
    n_rows = B * T
    t_split = 26

    def phase_gather(t0, t1):
        idx = x[:, t0:t1].reshape(_NW, (B * (t1 - t0)) // (_NW * _CHUNK),
                                  _CHUNK)
        oidx = jnp.asarray(_time_major_oidx(B, t0, t1))
        e = _sc_gather(embed_table, idx, oidx, B * (t1 - t0), E)
        return e.reshape((t1 - t0) // 2, B, 2 * E)

    e2_a = phase_gather(0, t_split)
    e2_b = phase_gather(t_split, T)

    zeros = jnp.zeros((E, E), jnp.float32)
    w1e = jnp.concatenate([W_ih.T, zeros], axis=0)
    w1o = jnp.concatenate([zeros, W_ih.T], axis=0)
    bh = b_h.reshape(1, E)
    bun = b_un.reshape(1, A)
    s0 = jnp.zeros((B, E), jnp.float32)
    y5_a, s_a = _rnn_unembed(e2_a, s0, w1e, w1o, W_hh.T, bh, W_un.T, bun)
    y5_b, _ = _rnn_unembed(e2_b, s_a, w1e, w1o, W_hh.T, bh, W_un.T, bun)
    ya = y5_a.reshape(B * t_split, A)
    yb = y5_b.reshape(B * (T - t_split), A)
    oq_a = jnp.asarray(_unscramble_oidx(B, T, 0, B * t_split))
    oq_b = jnp.asarray(_unscramble_oidx(B, T, B * t_split, n_rows))
    out_q = _sc_permute(ya, yb, oq_a, oq_b, n_rows, A)
    return jnp.transpose(out_q.reshape(T, B, A), (1, 0, 2))
